# baseline (speedup 1.0000x reference)
import jax
import jax.numpy as jnp
from jax import lax
from jax.experimental import pallas as pl
from jax.experimental.pallas import tpu as pltpu
from jax.experimental.pallas import tpu_sc as plsc

B, N, Q = 16, 2048, 128
NC, NS, L = 2, 16, 16
QH = Q // 2
NG = QH // 4
N_SC = 1024
N_TC = N - N_SC
CTC = N_TC // 128


def _sc_body(xs_hbm, ys_hbm, samples_hbm, out_hbm, xs_v, ys_v, samp_v, pool_v):
    c = lax.axis_index("c")
    s = lax.axis_index("s")
    b = (NS // 2) * c + s // 2
    h = s % 2

    pltpu.sync_copy(xs_hbm.at[b, pl.ds(0, N_SC)], xs_v)
    pltpu.sync_copy(ys_hbm.at[b, pl.ds(0, N_SC)], ys_v)
    pltpu.sync_copy(samples_hbm.at[pl.ds(h * QH, QH)], samp_v)

    t = [samp_v[pl.ds(j * L, L)] for j in range(QH // L)]

    def point_step(i, acc):
        xv = xs_v[pl.ds(i * L, L)]
        yv = ys_v[pl.ds(i * L, L)]
        for u in range(L):
            x = xv[u]
            y = yv[u]
            acc = tuple(
                a + jnp.maximum(jnp.minimum(tj - x, y - tj), 0.0)
                for a, tj in zip(acc, t)
            )
        return acc

    acc0 = tuple(jnp.zeros((L,), jnp.float32) for _ in range(QH // L))
    acc = lax.fori_loop(0, N_SC // L, point_step, acc0)
    for j in range(QH // L):
        pool_v[pl.ds(j * L, L)] = acc[j]

    pltpu.sync_copy(pool_v, out_hbm.at[b, pl.ds(h * QH, QH)])


def _tc_pool_body(xs_ref, ys_ref, samp_ref, out_ref):
    for bi in range(8):
        rows = []
        for g in range(16):
            sg = samp_ref[g]
            acc = jnp.zeros((8, 128), jnp.float32)
            for c in range(CTC):
                xr = xs_ref[bi, pl.ds(c * 128, 128)]
                yr = ys_ref[bi, pl.ds(c * 128, 128)]
                xb = jnp.broadcast_to(xr[None, :], (8, 128))
                yb = jnp.broadcast_to(yr[None, :], (8, 128))
                acc = acc + jnp.maximum(jnp.minimum(sg - xb, yb - sg), 0.0)
            rows.append(jnp.sum(acc, axis=1))
        out_ref[bi, :] = jnp.concatenate(rows, axis=0)


def _tc_combine_body(sc_ref, tc_ref, w_ref, b_ref, out_ref):
    pooled = sc_ref[...] + tc_ref[...]
    z = lax.dot_general(pooled, w_ref[...], (((1,), (1,)), ((), ())),
                        precision=lax.Precision.HIGHEST,
                        preferred_element_type=jnp.float32)
    out_ref[...] = jnp.maximum(z + b_ref[...], 0.0)


def kernel(diagram, samples, rho_w, rho_b):
    xs = diagram[..., 0]
    ys = diagram[..., 1]

    pooled_sc = pl.kernel(
        _sc_body,
        out_type=jax.ShapeDtypeStruct((B, Q), jnp.float32),
        mesh=plsc.VectorSubcoreMesh(core_axis_name="c", subcore_axis_name="s",
                                    num_cores=NC, num_subcores=NS),
        scratch_types=[
            pltpu.VMEM((N_SC,), jnp.float32),
            pltpu.VMEM((N_SC,), jnp.float32),
            pltpu.VMEM((QH,), jnp.float32),
            pltpu.VMEM((QH,), jnp.float32),
        ],
    )(xs, ys, samples)

    xs_tc = xs[:, N_SC:]
    ys_tc = ys[:, N_SC:]
    samples_bc = jnp.broadcast_to(samples.reshape(16, 8, 1), (16, 8, 128))

    pooled_tc = pl.pallas_call(
        _tc_pool_body,
        grid=(B // 8,),
        in_specs=[
            pl.BlockSpec((8, N_TC), lambda b: (b, 0)),
            pl.BlockSpec((8, N_TC), lambda b: (b, 0)),
            pl.BlockSpec((16, 8, 128), lambda b: (0, 0, 0)),
        ],
        out_specs=pl.BlockSpec((8, Q), lambda b: (b, 0)),
        out_shape=jax.ShapeDtypeStruct((B, Q), jnp.float32),
    )(xs_tc, ys_tc, samples_bc)

    out = pl.pallas_call(
        _tc_combine_body,
        in_specs=[
            pl.BlockSpec((B, Q), lambda: (0, 0)),
            pl.BlockSpec((B, Q), lambda: (0, 0)),
            pl.BlockSpec((Q, Q), lambda: (0, 0)),
            pl.BlockSpec((1, Q), lambda: (0, 0)),
        ],
        out_specs=pl.BlockSpec((B, Q), lambda: (0, 0)),
        out_shape=jax.ShapeDtypeStruct((B, Q), jnp.float32),
    )(pooled_sc, pooled_tc, rho_w, rho_b.reshape(1, Q))
    return out


# speedup vs baseline: 1.3035x; 1.3035x over previous
import jax
import jax.numpy as jnp
from jax import lax
from jax.experimental import pallas as pl
from jax.experimental.pallas import tpu as pltpu
from jax.experimental.pallas import tpu_sc as plsc

B, N, Q = 16, 2048, 128
NC, NS, L = 2, 16, 16
QH = Q // 2
NG = QH // 4
N_SC = 1024
N_TC = N - N_SC
CTC = N_TC // 128


def _sc_body(xs_hbm, ys_hbm, samples_hbm, out_hbm, xs_v, ys_v, samp_v, pool_v):
    c = lax.axis_index("c")
    s = lax.axis_index("s")
    b = (NS // 2) * c + s // 2
    h = s % 2

    pltpu.sync_copy(xs_hbm.at[b, pl.ds(0, N_SC)], xs_v.at[pl.ds(0, N_SC)])
    pltpu.sync_copy(ys_hbm.at[b, pl.ds(0, N_SC)], ys_v.at[pl.ds(0, N_SC)])
    pltpu.sync_copy(samples_hbm.at[pl.ds(h * QH, QH)], samp_v)

    t = [samp_v[pl.ds(j * L, L)] for j in range(QH // L)]

    def point_step(i, acc):
        xv = xs_v[pl.ds(i * 8, L)]
        yv = ys_v[pl.ds(i * 8, L)]
        for u in range(8):
            x = xv[u]
            y = yv[u]
            acc = tuple(
                a + jnp.maximum(jnp.minimum(tj - x, y - tj), 0.0)
                for a, tj in zip(acc, t)
            )
        return acc

    acc0 = tuple(jnp.zeros((L,), jnp.float32) for _ in range(QH // L))
    acc = lax.fori_loop(0, N_SC // 8, point_step, acc0)
    for j in range(QH // L):
        pool_v[pl.ds(j * L, L)] = acc[j]

    pltpu.sync_copy(pool_v, out_hbm.at[b, pl.ds(h * QH, QH)])


def _tc_pool_body(xs_ref, ys_ref, samp_ref, out_ref):
    for bi in range(8):
        rows = []
        for g in range(16):
            sg = samp_ref[g]
            acc = jnp.zeros((8, 128), jnp.float32)
            for c in range(CTC):
                xr = xs_ref[bi, pl.ds(c * 128, 128)]
                yr = ys_ref[bi, pl.ds(c * 128, 128)]
                xb = jnp.broadcast_to(xr[None, :], (8, 128))
                yb = jnp.broadcast_to(yr[None, :], (8, 128))
                acc = acc + jnp.maximum(jnp.minimum(sg - xb, yb - sg), 0.0)
            rows.append(jnp.sum(acc, axis=1))
        out_ref[bi, :] = jnp.concatenate(rows, axis=0)


def _tc_combine_body(sc_ref, tc_ref, w_ref, b_ref, out_ref):
    pooled = sc_ref[...] + tc_ref[...]
    z = lax.dot_general(pooled, w_ref[...], (((1,), (1,)), ((), ())),
                        precision=lax.Precision.HIGHEST,
                        preferred_element_type=jnp.float32)
    out_ref[...] = jnp.maximum(z + b_ref[...], 0.0)


def kernel(diagram, samples, rho_w, rho_b):
    xs = diagram[..., 0]
    ys = diagram[..., 1]

    pooled_sc = pl.kernel(
        _sc_body,
        out_type=jax.ShapeDtypeStruct((B, Q), jnp.float32),
        mesh=plsc.VectorSubcoreMesh(core_axis_name="c", subcore_axis_name="s",
                                    num_cores=NC, num_subcores=NS),
        scratch_types=[
            pltpu.VMEM((N_SC + L,), jnp.float32),
            pltpu.VMEM((N_SC + L,), jnp.float32),
            pltpu.VMEM((QH,), jnp.float32),
            pltpu.VMEM((QH,), jnp.float32),
        ],
    )(xs, ys, samples)

    xs_tc = xs[:, N_SC:]
    ys_tc = ys[:, N_SC:]
    samples_bc = jnp.broadcast_to(samples.reshape(16, 8, 1), (16, 8, 128))

    pooled_tc = pl.pallas_call(
        _tc_pool_body,
        grid=(B // 8,),
        in_specs=[
            pl.BlockSpec((8, N_TC), lambda b: (b, 0)),
            pl.BlockSpec((8, N_TC), lambda b: (b, 0)),
            pl.BlockSpec((16, 8, 128), lambda b: (0, 0, 0)),
        ],
        out_specs=pl.BlockSpec((8, Q), lambda b: (b, 0)),
        out_shape=jax.ShapeDtypeStruct((B, Q), jnp.float32),
    )(xs_tc, ys_tc, samples_bc)

    out = pl.pallas_call(
        _tc_combine_body,
        in_specs=[
            pl.BlockSpec((B, Q), lambda: (0, 0)),
            pl.BlockSpec((B, Q), lambda: (0, 0)),
            pl.BlockSpec((Q, Q), lambda: (0, 0)),
            pl.BlockSpec((1, Q), lambda: (0, 0)),
        ],
        out_specs=pl.BlockSpec((B, Q), lambda: (0, 0)),
        out_shape=jax.ShapeDtypeStruct((B, Q), jnp.float32),
    )(pooled_sc, pooled_tc, rho_w, rho_b.reshape(1, Q))
    return out


# speedup vs baseline: 1.3194x; 1.0122x over previous
import jax
import jax.numpy as jnp
from jax import lax
from jax.experimental import pallas as pl
from jax.experimental.pallas import tpu as pltpu
from jax.experimental.pallas import tpu_sc as plsc

B, N, Q = 16, 2048, 128
NC, NS, L = 2, 16, 16
QH = Q // 2
NG = QH // 4
N_SC = 896
N_TC = N - N_SC
CTC = N_TC // 128


def _sc_body(xs_hbm, ys_hbm, samples_hbm, out_hbm, xs_v, ys_v, samp_v, pool_v):
    c = lax.axis_index("c")
    s = lax.axis_index("s")
    b = (NS // 2) * c + s // 2
    h = s % 2

    pltpu.sync_copy(xs_hbm.at[b, pl.ds(0, N_SC)], xs_v.at[pl.ds(0, N_SC)])
    pltpu.sync_copy(ys_hbm.at[b, pl.ds(0, N_SC)], ys_v.at[pl.ds(0, N_SC)])
    pltpu.sync_copy(samples_hbm.at[pl.ds(h * QH, QH)], samp_v)

    t = [samp_v[pl.ds(j * L, L)] for j in range(QH // L)]

    def point_step(i, acc):
        xv = xs_v[pl.ds(i * 8, L)]
        yv = ys_v[pl.ds(i * 8, L)]
        for u in range(8):
            x = xv[u]
            y = yv[u]
            acc = tuple(
                a + jnp.maximum(jnp.minimum(tj - x, y - tj), 0.0)
                for a, tj in zip(acc, t)
            )
        return acc

    acc0 = tuple(jnp.zeros((L,), jnp.float32) for _ in range(QH // L))
    acc = lax.fori_loop(0, N_SC // 8, point_step, acc0)
    for j in range(QH // L):
        pool_v[pl.ds(j * L, L)] = acc[j]

    pltpu.sync_copy(pool_v, out_hbm.at[b, pl.ds(h * QH, QH)])


def _tc_pool_body(xs_ref, ys_ref, samp_ref, out_ref):
    for bi in range(8):
        rows = []
        for g in range(16):
            sg = samp_ref[g]
            acc = jnp.zeros((8, 128), jnp.float32)
            for c in range(CTC):
                xr = xs_ref[bi, pl.ds(N_SC + c * 128, 128)]
                yr = ys_ref[bi, pl.ds(N_SC + c * 128, 128)]
                xb = jnp.broadcast_to(xr[None, :], (8, 128))
                yb = jnp.broadcast_to(yr[None, :], (8, 128))
                acc = acc + jnp.maximum(jnp.minimum(sg - xb, yb - sg), 0.0)
            rows.append(jnp.sum(acc, axis=1))
        out_ref[bi, :] = jnp.concatenate(rows, axis=0)


def _tc_combine_body(sc_ref, tc_ref, w_ref, b_ref, out_ref):
    pooled = sc_ref[...] + tc_ref[...]
    z = lax.dot_general(pooled, w_ref[...], (((1,), (1,)), ((), ())),
                        precision=lax.Precision.HIGHEST,
                        preferred_element_type=jnp.float32)
    out_ref[...] = jnp.maximum(z + b_ref[...], 0.0)


def kernel(diagram, samples, rho_w, rho_b):
    xs = diagram[..., 0]
    ys = diagram[..., 1]

    pooled_sc = pl.kernel(
        _sc_body,
        out_type=jax.ShapeDtypeStruct((B, Q), jnp.float32),
        mesh=plsc.VectorSubcoreMesh(core_axis_name="c", subcore_axis_name="s",
                                    num_cores=NC, num_subcores=NS),
        scratch_types=[
            pltpu.VMEM((N_SC + L,), jnp.float32),
            pltpu.VMEM((N_SC + L,), jnp.float32),
            pltpu.VMEM((QH,), jnp.float32),
            pltpu.VMEM((QH,), jnp.float32),
        ],
    )(xs, ys, samples)

    samples_bc = jnp.broadcast_to(samples.reshape(16, 8, 1), (16, 8, 128))

    pooled_tc = pl.pallas_call(
        _tc_pool_body,
        grid=(B // 8,),
        in_specs=[
            pl.BlockSpec((8, N), lambda b: (b, 0)),
            pl.BlockSpec((8, N), lambda b: (b, 0)),
            pl.BlockSpec((16, 8, 128), lambda b: (0, 0, 0)),
        ],
        out_specs=pl.BlockSpec((8, Q), lambda b: (b, 0)),
        out_shape=jax.ShapeDtypeStruct((B, Q), jnp.float32),
    )(xs, ys, samples_bc)

    out = pl.pallas_call(
        _tc_combine_body,
        in_specs=[
            pl.BlockSpec((B, Q), lambda: (0, 0)),
            pl.BlockSpec((B, Q), lambda: (0, 0)),
            pl.BlockSpec((Q, Q), lambda: (0, 0)),
            pl.BlockSpec((1, Q), lambda: (0, 0)),
        ],
        out_specs=pl.BlockSpec((B, Q), lambda: (0, 0)),
        out_shape=jax.ShapeDtypeStruct((B, Q), jnp.float32),
    )(pooled_sc, pooled_tc, rho_w, rho_b.reshape(1, Q))
    return out
